# Initial kernel scaffold; baseline (speedup 1.0000x reference)
import jax
import jax.numpy as jnp
from jax.experimental import pallas as pl


def kernel(pred_logits, pred_boxes, target_sizes, pred_count):
    raise NotImplementedError("write your pallas kernel here")


# speedup vs baseline: 1.0003x; 1.0003x over previous
import jax
import jax.numpy as jnp
from jax.experimental import pallas as pl


def _sigmoid_kernel(x_ref, o_ref):
    o_ref[...] = jax.nn.sigmoid(x_ref[...])


def kernel(pred_logits, pred_boxes, target_sizes, pred_count):
    N, Nq, Nc = pred_logits.shape
    prob = pl.pallas_call(
        _sigmoid_kernel,
        out_shape=jax.ShapeDtypeStruct((N, Nq, Nc), jnp.float32),
        grid=(N,),
        in_specs=[pl.BlockSpec((1, Nq, Nc), lambda i: (i, 0, 0))],
        out_specs=pl.BlockSpec((1, Nq, Nc), lambda i: (i, 0, 0)),
    )(pred_logits)
    flat = prob.reshape(N, Nq * Nc)
    topk_values, topk_indexes = jax.lax.top_k(flat, Nq)
    scores = topk_values
    topk_boxes = topk_indexes // Nc
    labels = topk_indexes % Nc
    c = pred_boxes[..., 0]
    l = pred_boxes[..., 1]
    boxes = jnp.stack([c - 0.5 * l, c + 0.5 * l], axis=-1)
    raw_boxes = boxes
    boxes = jnp.clip(boxes, 0.0, 1.0)
    gather_idx = jnp.broadcast_to(topk_boxes[:, :, None], (N, Nq, 2))
    boxes = jnp.take_along_axis(boxes, gather_idx, axis=1)
    scale_fct = jnp.stack([target_sizes, target_sizes], axis=1)
    boxes = boxes * scale_fct[:, None, :]
    eseq_lens = jnp.maximum(jnp.argmax(pred_count, axis=-1), 1)
    return (scores, labels, boxes, raw_boxes, topk_boxes, eseq_lens, topk_indexes)


# speedup vs baseline: 5.5892x; 5.5877x over previous
import jax
import jax.numpy as jnp
from jax import lax
from jax.experimental import pallas as pl
from jax.experimental.pallas import tpu as pltpu
from jax.experimental.pallas import tpu_sc as plsc

N, NQ, NC = 64, 1000, 1000
NE = NQ * NC
K = NQ
CAP = 2048
NBINS = 16384
BIN_SHIFT = 18
CHUNK = 20000
NCHUNKS = NE // CHUNK
NWORKERS = 32
ROWS_PER_W = N // NWORKERS
PAD_IDX = NE
RBLK = 128


def _sc_select_body(x_hbm, out_x_hbm, out_i_hbm, buf, hist, cand_v, cand_i):
    wid = lax.axis_index("s") * 2 + lax.axis_index("c")
    lanes = lax.iota(jnp.int32, 16)
    ones_i = jnp.ones((16,), jnp.int32)
    zeros_i = jnp.zeros((16,), jnp.int32)
    true_m = jnp.ones((16,), jnp.bool_)
    minint = jnp.full((16,), jnp.int32(-2147483648), jnp.int32)

    def monotone_bin(x):
        b = lax.bitcast_convert_type(x, jnp.int32)
        m = lax.shift_right_arithmetic(b, 31)
        u = lax.bitwise_xor(b, lax.bitwise_or(m, minint))
        return lax.shift_right_logical(u, BIN_SHIFT)

    for rr in range(ROWS_PER_W):
        r = wid * ROWS_PER_W + rr

        def zero_hist(i, _):
            hist[pl.ds(i * 16, 16)] = zeros_i
            return 0

        lax.fori_loop(0, NBINS // 16, zero_hist, 0)

        def pass1_chunk(c, _):
            pltpu.sync_copy(x_hbm.at[pl.ds(r * NE + c * CHUNK, CHUNK)], buf)

            def it(i, _):
                x = buf[pl.ds(i * 16, 16)]
                bn = monotone_bin(x)
                plsc.addupdate_scatter(hist, [bn], ones_i, mask=true_m)
                return 0

            lax.fori_loop(0, CHUNK // 16, it, 0)
            return 0

        lax.fori_loop(0, NCHUNKS, pass1_chunk, 0)

        def scan_bins(j, carry):
            run, best = carry
            jj = NBINS // 16 - 1 - j
            v = hist[pl.ds(jj * 16, 16)]
            pre = plsc.cumsum(v)
            tot = jnp.max(pre)
            suf = (run + tot) - pre + v
            mask = suf >= K
            bvec = jnp.where(mask, jnp.broadcast_to(jj * 16, (16,)) + lanes,
                             jnp.full((16,), -1, jnp.int32))
            best = jnp.maximum(best, jnp.max(bvec))
            return (run + tot, best)

        _, b1 = lax.fori_loop(0, NBINS // 16, scan_bins,
                              (jnp.int32(0), jnp.int32(-1)))
        b_star = jnp.maximum(b1 - 1, 0)
        bsv = jnp.broadcast_to(b_star, (16,))

        def prefill(i, _):
            cand_v[pl.ds(i * 16, 16)] = jnp.full((16,), -jnp.inf, jnp.float32)
            cand_i[pl.ds(i * 16, 16)] = jnp.full((16,), PAD_IDX, jnp.int32)
            return 0

        lax.fori_loop(0, CAP // 16, prefill, 0)

        def pass2_chunk(c, off):
            pltpu.sync_copy(x_hbm.at[pl.ds(r * NE + c * CHUNK, CHUNK)], buf)

            def it(i, off):
                x = buf[pl.ds(i * 16, 16)]
                bn = monotone_bin(x)
                m = bn >= bsv
                mi = jnp.where(m, ones_i, zeros_i)
                pos = off + plsc.cumsum(mi) - 1
                m2 = jnp.logical_and(m, pos < CAP)
                plsc.store_scatter(cand_v, [pos], x, mask=m2)
                idxv = jnp.broadcast_to(c * CHUNK + i * 16, (16,)) + lanes
                plsc.store_scatter(cand_i, [pos], idxv, mask=m2)
                return off + plsc.all_reduce_population_count(m)

            return lax.fori_loop(0, CHUNK // 16, it, off)

        lax.fori_loop(0, NCHUNKS, pass2_chunk, zeros_i)

        pltpu.sync_copy(cand_v, out_x_hbm.at[pl.ds(r * CAP, CAP)])
        pltpu.sync_copy(cand_i, out_i_hbm.at[pl.ds(r * CAP, CAP)])


def _sc_select(logits_flat):
    mesh = plsc.VectorSubcoreMesh(core_axis_name="c", subcore_axis_name="s")
    kern = pl.kernel(
        _sc_select_body,
        out_type=[
            jax.ShapeDtypeStruct((N * CAP,), jnp.float32),
            jax.ShapeDtypeStruct((N * CAP,), jnp.int32),
        ],
        mesh=mesh,
        scratch_types=[
            pltpu.VMEM((CHUNK,), jnp.float32),
            pltpu.VMEM((NBINS,), jnp.int32),
            pltpu.VMEM((CAP,), jnp.float32),
            pltpu.VMEM((CAP,), jnp.int32),
        ],
        compiler_params=pltpu.CompilerParams(needs_layout_passes=False),
    )
    return kern(logits_flat)


def _rank_body(xr_ref, xT_ref, iT_ref, c_ref, l_ref, ts_ref,
               scores_ref, labels_ref, tb_ref, ti_ref,
               b0_ref, b1_ref, r0_ref, r1_ref):
    p_row = jax.nn.sigmoid(xr_ref[0])
    p_col = jax.nn.sigmoid(xT_ref[0])

    lane_pos = lax.broadcasted_iota(jnp.int32, (RBLK, CAP), 1)
    blocks = []
    for ib in range(CAP // RBLK):
        pc = p_col[ib * RBLK:(ib + 1) * RBLK, :]
        ipos = (lax.broadcasted_iota(jnp.int32, (RBLK, CAP), 0)
                + ib * RBLK)
        gt = p_row > pc
        tie = jnp.logical_and(p_row == pc, lane_pos < ipos)
        e = jnp.where(jnp.logical_or(gt, tie), 1.0, 0.0)
        blocks.append(jnp.sum(e, axis=1, keepdims=True))
    rank_col = jnp.concatenate(blocks, axis=0).astype(jnp.int32)

    idx_col = iT_ref[0]
    q_col = idx_col // NC
    lab_col = idx_col - q_col * NC

    cc = c_ref[0]
    ll = l_ref[0]
    s0 = cc - 0.5 * ll
    s1 = cc + 0.5 * ll
    r0_ref[...] = s0.reshape(1, 1, NQ)
    r1_ref[...] = s1.reshape(1, 1, NQ)
    cl0 = jnp.clip(s0, 0.0, 1.0)
    cl1 = jnp.clip(s1, 0.0, 1.0)

    qiota = lax.broadcasted_iota(jnp.int32, (CAP, NQ), 1)
    oq = jnp.where(q_col == qiota, 1.0, 0.0)
    dn = (((1,), (1,)), ((), ()))
    bc0 = lax.dot_general(oq, cl0, dn, precision=lax.Precision.HIGHEST)
    bc1 = lax.dot_general(oq, cl1, dn, precision=lax.Precision.HIGHEST)

    vals = jnp.concatenate(
        [p_col, idx_col.astype(jnp.float32), lab_col.astype(jnp.float32),
         q_col.astype(jnp.float32), bc0, bc1], axis=1)
    kiota = lax.broadcasted_iota(jnp.int32, (CAP, NQ), 1)
    oh = jnp.where(rank_col == kiota, 1.0, 0.0)
    dn0 = (((0,), (0,)), ((), ()))
    out6 = lax.dot_general(vals, oh, dn0,
                           precision=lax.Precision.HIGHEST)

    ts = ts_ref[0, 0, 0]
    scores_ref[...] = out6[0:1, :].reshape(1, 1, NQ)
    ti_ref[...] = out6[1:2, :].astype(jnp.int32).reshape(1, 1, NQ)
    labels_ref[...] = out6[2:3, :].astype(jnp.int32).reshape(1, 1, NQ)
    tb_ref[...] = out6[3:4, :].astype(jnp.int32).reshape(1, 1, NQ)
    b0_ref[...] = (out6[4:5, :] * ts).reshape(1, 1, NQ)
    b1_ref[...] = (out6[5:6, :] * ts).reshape(1, 1, NQ)


def _eseq_body(pc_ref, out_ref):
    pc = pc_ref[...]
    mx = jnp.max(pc, axis=1, keepdims=True)
    ii = lax.broadcasted_iota(jnp.int32, pc.shape, 1)
    first = jnp.min(jnp.where(pc == mx, ii, 101), axis=1, keepdims=True)
    out_ref[...] = jnp.maximum(first, 1)


def kernel(pred_logits, pred_boxes, target_sizes, pred_count):
    logits_flat = pred_logits.reshape(N * NE)
    cand_x_flat, cand_i_flat = _sc_select(logits_flat)

    cand_x = cand_x_flat.reshape(N, 1, CAP)
    cand_xT = cand_x_flat.reshape(N, CAP, 1)
    cand_iT = cand_i_flat.reshape(N, CAP, 1)
    c3 = pred_boxes[..., 0].reshape(N, 1, NQ)
    l3 = pred_boxes[..., 1].reshape(N, 1, NQ)
    ts3 = target_sizes.reshape(N, 1, 1)

    row = lambda i: (i, 0, 0)
    f32 = jnp.float32
    i32 = jnp.int32
    outs = pl.pallas_call(
        _rank_body,
        grid=(N,),
        in_specs=[
            pl.BlockSpec((1, 1, CAP), row),
            pl.BlockSpec((1, CAP, 1), row),
            pl.BlockSpec((1, CAP, 1), row),
            pl.BlockSpec((1, 1, NQ), row),
            pl.BlockSpec((1, 1, NQ), row),
            pl.BlockSpec((1, 1, 1), row),
        ],
        out_specs=[
            pl.BlockSpec((1, 1, NQ), row), pl.BlockSpec((1, 1, NQ), row),
            pl.BlockSpec((1, 1, NQ), row), pl.BlockSpec((1, 1, NQ), row),
            pl.BlockSpec((1, 1, NQ), row), pl.BlockSpec((1, 1, NQ), row),
            pl.BlockSpec((1, 1, NQ), row), pl.BlockSpec((1, 1, NQ), row),
        ],
        out_shape=[
            jax.ShapeDtypeStruct((N, 1, NQ), f32),
            jax.ShapeDtypeStruct((N, 1, NQ), i32),
            jax.ShapeDtypeStruct((N, 1, NQ), i32),
            jax.ShapeDtypeStruct((N, 1, NQ), i32),
            jax.ShapeDtypeStruct((N, 1, NQ), f32),
            jax.ShapeDtypeStruct((N, 1, NQ), f32),
            jax.ShapeDtypeStruct((N, 1, NQ), f32),
            jax.ShapeDtypeStruct((N, 1, NQ), f32),
        ],
    )(cand_x, cand_xT, cand_iT, c3, l3, ts3)
    scores, labels, tb, ti, b0, b1, r0, r1 = outs

    eseq = pl.pallas_call(
        _eseq_body,
        out_shape=jax.ShapeDtypeStruct((N, 1), jnp.int32),
    )(pred_count)

    scores = scores.reshape(N, NQ)
    labels = labels.reshape(N, NQ)
    topk_boxes = tb.reshape(N, NQ)
    topk_indexes = ti.reshape(N, NQ)
    boxes = jnp.stack([b0.reshape(N, NQ), b1.reshape(N, NQ)], axis=-1)
    raw_boxes = jnp.stack([r0.reshape(N, NQ), r1.reshape(N, NQ)], axis=-1)
    eseq_lens = eseq.reshape(N)
    return (scores, labels, boxes, raw_boxes, topk_boxes, eseq_lens,
            topk_indexes)


# speedup vs baseline: 5.9280x; 1.0606x over previous
import jax
import jax.numpy as jnp
from jax import lax
from jax.experimental import pallas as pl
from jax.experimental.pallas import tpu as pltpu
from jax.experimental.pallas import tpu_sc as plsc

N, NQ, NC = 64, 1000, 1000
NE = NQ * NC
K = NQ
CAP = 2048
NBINS = 16384
BIN_SHIFT = 18
CHUNK = 40000
NCHUNKS = NE // CHUNK
UNROLL = 4
NWORKERS = 32
ROWS_PER_W = N // NWORKERS
PAD_IDX = NE
RBLK = 128


def _sc_select_body(x_hbm, out_x_hbm, out_i_hbm, buf0, buf1, hist,
                    cand_v, cand_i, sem0, sem1):
    wid = lax.axis_index("s") * 2 + lax.axis_index("c")
    lanes = lax.iota(jnp.int32, 16)
    ones_i = jnp.ones((16,), jnp.int32)
    zeros_i = jnp.zeros((16,), jnp.int32)
    minint = jnp.full((16,), jnp.int32(-2147483648), jnp.int32)

    def monotone_bin(x):
        b = lax.bitcast_convert_type(x, jnp.int32)
        m = lax.shift_right_arithmetic(b, 31)
        u = lax.bitwise_xor(b, lax.bitwise_or(m, minint))
        return lax.shift_right_logical(u, BIN_SHIFT)

    def chunk_src(r, c):
        return x_hbm.at[pl.ds(r * NE + c * CHUNK, CHUNK)]

    def dma(r, c, buf, sem):
        return pltpu.make_async_copy(chunk_src(r, c), buf, sem)

    def ring_loop(r, compute, carry_init):
        dma(r, 0, buf0, sem0).start()

        def body(c, carry):
            even = lax.rem(c, 2) == 0

            def step(buf_a, sem_a, buf_b, sem_b, carry):
                @pl.when(c + 1 < NCHUNKS)
                def _():
                    dma(r, c + 1, buf_b, sem_b).start()

                dma(r, c, buf_a, sem_a).wait()
                return compute(c, buf_a, carry)

            return lax.cond(
                even,
                lambda cr: step(buf0, sem0, buf1, sem1, cr),
                lambda cr: step(buf1, sem1, buf0, sem0, cr),
                carry,
            )

        return lax.fori_loop(0, NCHUNKS, body, carry_init)

    for rr in range(ROWS_PER_W):
        r = wid * ROWS_PER_W + rr

        def zero_hist(i, _):
            hist[pl.ds(i * 16, 16)] = zeros_i
            return 0

        lax.fori_loop(0, NBINS // 16, zero_hist, 0)

        def p1_compute(c, b, _):
            def it(i, _):
                for u in range(UNROLL):
                    x = b[pl.ds(i * (16 * UNROLL) + u * 16, 16)]
                    bn = monotone_bin(x)
                    plsc.addupdate_scatter(hist, [bn], ones_i)
                return 0

            lax.fori_loop(0, CHUNK // (16 * UNROLL), it, 0)
            return 0

        ring_loop(r, p1_compute, 0)

        def scan_bins(j, carry):
            run, best = carry
            jj = NBINS // 16 - 1 - j
            v = hist[pl.ds(jj * 16, 16)]
            pre = plsc.cumsum(v)
            tot = jnp.max(pre)
            suf = (run + tot) - pre + v
            mask = suf >= K
            bvec = jnp.where(mask, jnp.broadcast_to(jj * 16, (16,)) + lanes,
                             jnp.full((16,), -1, jnp.int32))
            best = jnp.maximum(best, jnp.max(bvec))
            return (run + tot, best)

        _, b1 = lax.fori_loop(0, NBINS // 16, scan_bins,
                              (jnp.int32(0), jnp.int32(-1)))
        b_star = jnp.maximum(b1 - 1, 0)
        bsv = jnp.broadcast_to(b_star, (16,))

        def prefill(i, _):
            cand_v[pl.ds(i * 16, 16)] = jnp.full((16,), -jnp.inf, jnp.float32)
            cand_i[pl.ds(i * 16, 16)] = jnp.full((16,), PAD_IDX, jnp.int32)
            return 0

        lax.fori_loop(0, CAP // 16, prefill, 0)

        def p2_compute(c, b, off):
            def it(i, off):
                for u in range(UNROLL):
                    base = i * (16 * UNROLL) + u * 16
                    x = b[pl.ds(base, 16)]
                    bn = monotone_bin(x)
                    m = bn >= bsv
                    mi = jnp.where(m, ones_i, zeros_i)
                    pos = off + plsc.cumsum(mi) - 1
                    m2 = jnp.logical_and(m, pos < CAP)
                    plsc.store_scatter(cand_v, [pos], x, mask=m2)
                    idxv = jnp.broadcast_to(c * CHUNK + base, (16,)) + lanes
                    plsc.store_scatter(cand_i, [pos], idxv, mask=m2)
                    off = off + plsc.all_reduce_population_count(m)
                return off

            return lax.fori_loop(0, CHUNK // (16 * UNROLL), it, off)

        ring_loop(r, p2_compute, zeros_i)

        pltpu.sync_copy(cand_v, out_x_hbm.at[pl.ds(r * CAP, CAP)])
        pltpu.sync_copy(cand_i, out_i_hbm.at[pl.ds(r * CAP, CAP)])


def _sc_select(logits_flat):
    mesh = plsc.VectorSubcoreMesh(core_axis_name="c", subcore_axis_name="s")
    kern = pl.kernel(
        _sc_select_body,
        out_type=[
            jax.ShapeDtypeStruct((N * CAP,), jnp.float32),
            jax.ShapeDtypeStruct((N * CAP,), jnp.int32),
        ],
        mesh=mesh,
        scratch_types=[
            pltpu.VMEM((CHUNK,), jnp.float32),
            pltpu.VMEM((CHUNK,), jnp.float32),
            pltpu.VMEM((NBINS,), jnp.int32),
            pltpu.VMEM((CAP,), jnp.float32),
            pltpu.VMEM((CAP,), jnp.int32),
            pltpu.SemaphoreType.DMA,
            pltpu.SemaphoreType.DMA,
        ],
        compiler_params=pltpu.CompilerParams(needs_layout_passes=False),
    )
    return kern(logits_flat)


def _rank_body(xr_ref, xT_ref, iT_ref, c_ref, l_ref, ts_ref,
               scores_ref, labels_ref, tb_ref, ti_ref,
               b0_ref, b1_ref, r0_ref, r1_ref):
    p_row = jax.nn.sigmoid(xr_ref[0])
    p_col = jax.nn.sigmoid(xT_ref[0])

    lane_pos = lax.broadcasted_iota(jnp.int32, (RBLK, CAP), 1)
    blocks = []
    for ib in range(CAP // RBLK):
        pc = p_col[ib * RBLK:(ib + 1) * RBLK, :]
        ipos = (lax.broadcasted_iota(jnp.int32, (RBLK, CAP), 0)
                + ib * RBLK)
        gt = p_row > pc
        tie = jnp.logical_and(p_row == pc, lane_pos < ipos)
        e = jnp.where(jnp.logical_or(gt, tie), 1.0, 0.0)
        blocks.append(jnp.sum(e, axis=1, keepdims=True))
    rank_col = jnp.concatenate(blocks, axis=0).astype(jnp.int32)

    idx_col = iT_ref[0]
    q_col = idx_col // NC
    lab_col = idx_col - q_col * NC

    cc = c_ref[0]
    ll = l_ref[0]
    s0 = cc - 0.5 * ll
    s1 = cc + 0.5 * ll
    r0_ref[...] = s0.reshape(1, 1, NQ)
    r1_ref[...] = s1.reshape(1, 1, NQ)
    cl0 = jnp.clip(s0, 0.0, 1.0)
    cl1 = jnp.clip(s1, 0.0, 1.0)

    qiota = lax.broadcasted_iota(jnp.int32, (CAP, NQ), 1)
    oq = jnp.where(q_col == qiota, 1.0, 0.0)
    dn = (((1,), (1,)), ((), ()))
    bc0 = lax.dot_general(oq, cl0, dn, precision=lax.Precision.HIGHEST)
    bc1 = lax.dot_general(oq, cl1, dn, precision=lax.Precision.HIGHEST)

    vals = jnp.concatenate(
        [p_col, idx_col.astype(jnp.float32), lab_col.astype(jnp.float32),
         q_col.astype(jnp.float32), bc0, bc1], axis=1)
    kiota = lax.broadcasted_iota(jnp.int32, (CAP, NQ), 1)
    oh = jnp.where(rank_col == kiota, 1.0, 0.0)
    dn0 = (((0,), (0,)), ((), ()))
    out6 = lax.dot_general(vals, oh, dn0,
                           precision=lax.Precision.HIGHEST)

    ts = ts_ref[0, 0, 0]
    scores_ref[...] = out6[0:1, :].reshape(1, 1, NQ)
    ti_ref[...] = out6[1:2, :].astype(jnp.int32).reshape(1, 1, NQ)
    labels_ref[...] = out6[2:3, :].astype(jnp.int32).reshape(1, 1, NQ)
    tb_ref[...] = out6[3:4, :].astype(jnp.int32).reshape(1, 1, NQ)
    b0_ref[...] = (out6[4:5, :] * ts).reshape(1, 1, NQ)
    b1_ref[...] = (out6[5:6, :] * ts).reshape(1, 1, NQ)


def _eseq_body(pc_ref, out_ref):
    pc = pc_ref[...]
    mx = jnp.max(pc, axis=1, keepdims=True)
    ii = lax.broadcasted_iota(jnp.int32, pc.shape, 1)
    first = jnp.min(jnp.where(pc == mx, ii, 101), axis=1, keepdims=True)
    out_ref[...] = jnp.maximum(first, 1)


def kernel(pred_logits, pred_boxes, target_sizes, pred_count):
    logits_flat = pred_logits.reshape(N * NE)
    cand_x_flat, cand_i_flat = _sc_select(logits_flat)

    cand_x = cand_x_flat.reshape(N, 1, CAP)
    cand_xT = cand_x_flat.reshape(N, CAP, 1)
    cand_iT = cand_i_flat.reshape(N, CAP, 1)
    c3 = pred_boxes[..., 0].reshape(N, 1, NQ)
    l3 = pred_boxes[..., 1].reshape(N, 1, NQ)
    ts3 = target_sizes.reshape(N, 1, 1)

    row = lambda i: (i, 0, 0)
    f32 = jnp.float32
    i32 = jnp.int32
    outs = pl.pallas_call(
        _rank_body,
        grid=(N,),
        in_specs=[
            pl.BlockSpec((1, 1, CAP), row),
            pl.BlockSpec((1, CAP, 1), row),
            pl.BlockSpec((1, CAP, 1), row),
            pl.BlockSpec((1, 1, NQ), row),
            pl.BlockSpec((1, 1, NQ), row),
            pl.BlockSpec((1, 1, 1), row),
        ],
        out_specs=[
            pl.BlockSpec((1, 1, NQ), row), pl.BlockSpec((1, 1, NQ), row),
            pl.BlockSpec((1, 1, NQ), row), pl.BlockSpec((1, 1, NQ), row),
            pl.BlockSpec((1, 1, NQ), row), pl.BlockSpec((1, 1, NQ), row),
            pl.BlockSpec((1, 1, NQ), row), pl.BlockSpec((1, 1, NQ), row),
        ],
        out_shape=[
            jax.ShapeDtypeStruct((N, 1, NQ), f32),
            jax.ShapeDtypeStruct((N, 1, NQ), i32),
            jax.ShapeDtypeStruct((N, 1, NQ), i32),
            jax.ShapeDtypeStruct((N, 1, NQ), i32),
            jax.ShapeDtypeStruct((N, 1, NQ), f32),
            jax.ShapeDtypeStruct((N, 1, NQ), f32),
            jax.ShapeDtypeStruct((N, 1, NQ), f32),
            jax.ShapeDtypeStruct((N, 1, NQ), f32),
        ],
    )(cand_x, cand_xT, cand_iT, c3, l3, ts3)
    scores, labels, tb, ti, b0, b1, r0, r1 = outs

    eseq = pl.pallas_call(
        _eseq_body,
        out_shape=jax.ShapeDtypeStruct((N, 1), jnp.int32),
    )(pred_count)

    scores = scores.reshape(N, NQ)
    labels = labels.reshape(N, NQ)
    topk_boxes = tb.reshape(N, NQ)
    topk_indexes = ti.reshape(N, NQ)
    boxes = jnp.stack([b0.reshape(N, NQ), b1.reshape(N, NQ)], axis=-1)
    raw_boxes = jnp.stack([r0.reshape(N, NQ), r1.reshape(N, NQ)], axis=-1)
    eseq_lens = eseq.reshape(N)
    return (scores, labels, boxes, raw_boxes, topk_boxes, eseq_lens,
            topk_indexes)
